# baseline (speedup 1.0000x reference)
import functools

import jax
import jax.numpy as jnp
from jax import lax
from jax.experimental import pallas as pl
from jax.experimental.pallas import tpu as pltpu
from jax.experimental.pallas import tpu_sc as plsc

_NUM_BLUEPRINTS = 100000
_NULL_IDX = _NUM_BLUEPRINTS
_D = 64
_NC = 2
_NS = 16
_NW = _NC * _NS
_BATCH = 4096
_SLOTS = 26
_BPW = _BATCH // _NW


@functools.partial(
    pl.kernel,
    out_type=jax.ShapeDtypeStruct((_SLOTS, _D, _BATCH), jnp.float32),
    mesh=plsc.VectorSubcoreMesh(core_axis_name="c", subcore_axis_name="s"),
    scratch_types=[
        pltpu.VMEM((_SLOTS, _BPW), jnp.int32),
        pltpu.VMEM((2, _BPW, _D), jnp.float32),
        pltpu.VMEM((2, _D, _BPW), jnp.float32),
        pltpu.SemaphoreType.DMA,
        pltpu.SemaphoreType.DMA,
        pltpu.SemaphoreType.DMA,
    ],
    compiler_params=pltpu.CompilerParams(use_tc_tiling_on_sc=False,
                                         needs_layout_passes=False),
)
def _sc_gather(idx_hbm, table_hbm, out_hbm, idx_v, g_v, gt_v, gsem, ssem0,
               ssem1):
    wid = lax.axis_index("s") * _NC + lax.axis_index("c")
    b0 = wid * _BPW

    pltpu.sync_copy(idx_hbm.at[:, pl.ds(b0, _BPW)], idx_v)

    def _remap(j, carry):
        for i in range(_BPW // 16):
            v = idx_v[j, pl.ds(i * 16, 16)]
            u = jnp.minimum(plsc.bitcast(v, jnp.uint32), jnp.uint32(_NULL_IDX))
            idx_v[j, pl.ds(i * 16, 16)] = plsc.bitcast(u, jnp.int32)
        return carry

    lax.fori_loop(0, _SLOTS, _remap, 0)

    iota16 = lax.iota(jnp.int32, 16)
    diag = [(iota16 + k) & 15 for k in range(16)]
    ssems = (ssem0, ssem1)

    def _fire_gather(s, p):
        pltpu.async_copy(table_hbm.at[idx_v.at[s]], g_v.at[p], gsem)

    def _wait_gather(s, p):
        pltpu.make_async_copy(table_hbm.at[idx_v.at[s]], g_v.at[p],
                              gsem).wait()

    def _fire_store(s, p):
        pltpu.async_copy(gt_v.at[p], out_hbm.at[s, :, pl.ds(b0, _BPW)],
                         ssems[p])

    def _wait_store(s, p):
        pltpu.make_async_copy(gt_v.at[p], out_hbm.at[s, :, pl.ds(b0, _BPW)],
                              ssems[p]).wait()

    def _transpose(p):
        def _tb(t, c2):
            for u in range(2):
                b0 = (t * 2 + u) * 16
                rowv = iota16 + b0
                for d0 in range(0, _D, 16):
                    loaded = []
                    for k in range(16):
                        colv = diag[k] + d0
                        loaded.append(
                            (colv, plsc.load_gather(g_v.at[p], [rowv, colv])))
                    for colv, v in loaded:
                        plsc.store_scatter(gt_v.at[p], [colv, rowv], v)
            return c2

        lax.fori_loop(0, _BPW // 32, _tb, 0)

    _fire_gather(0, 0)

    def _pipe(t, carry):
        for u in range(2):
            s = 2 * t + u
            p = u
            _wait_gather(s, p)

            @pl.when(s + 1 < _SLOTS)
            def _():
                _fire_gather(s + 1, 1 - p)

            @pl.when(s >= 2)
            def _():
                _wait_store(s - 2, p)

            _transpose(p)
            _fire_store(s, p)
        return carry

    lax.fori_loop(0, _SLOTS // 2, _pipe, 0)
    _wait_store(_SLOTS - 2, 0)
    _wait_store(_SLOTS - 1, 1)


def kernel(blueprint_indices, table):
    idx_t = blueprint_indices.T.astype(jnp.int32)
    p = _sc_gather(idx_t, table)
    return lax.transpose(p, (2, 0, 1))


# speedup vs baseline: 1.0197x; 1.0197x over previous
import functools

import jax
import jax.numpy as jnp
from jax import lax
from jax.experimental import pallas as pl
from jax.experimental.pallas import tpu as pltpu
from jax.experimental.pallas import tpu_sc as plsc

_NUM_BLUEPRINTS = 100000
_NULL_IDX = _NUM_BLUEPRINTS
_D = 64
_NC = 2
_NS = 16
_NW = _NC * _NS
_BATCH = 4096
_SLOTS = 26
_BPW = _BATCH // _NW


@functools.partial(
    pl.kernel,
    out_type=jax.ShapeDtypeStruct((_SLOTS, _D, _BATCH), jnp.float32),
    mesh=plsc.VectorSubcoreMesh(core_axis_name="c", subcore_axis_name="s"),
    scratch_types=[
        pltpu.VMEM((_SLOTS, _BPW), jnp.int32),
        pltpu.VMEM((2, _BPW, _D), jnp.float32),
        pltpu.VMEM((2, _D, _BPW), jnp.float32),
        pltpu.SemaphoreType.DMA,
        pltpu.SemaphoreType.DMA,
        pltpu.SemaphoreType.DMA,
    ],
    compiler_params=pltpu.CompilerParams(use_tc_tiling_on_sc=False,
                                         needs_layout_passes=False),
)
def _sc_gather(idx_hbm, table_hbm, out_hbm, idx_v, g_v, gt_v, gsem, ssem0,
               ssem1):
    wid = lax.axis_index("s") * _NC + lax.axis_index("c")
    b0 = wid * _BPW

    pltpu.sync_copy(idx_hbm.at[:, pl.ds(b0, _BPW)], idx_v)

    def _remap(j, carry):
        for i in range(_BPW // 16):
            v = idx_v[j, pl.ds(i * 16, 16)]
            u = jnp.minimum(plsc.bitcast(v, jnp.uint32), jnp.uint32(_NULL_IDX))
            idx_v[j, pl.ds(i * 16, 16)] = plsc.bitcast(u, jnp.int32)
        return carry

    lax.fori_loop(0, _SLOTS, _remap, 0)

    iota16 = lax.iota(jnp.int32, 16)
    diag = [(iota16 + k) & 15 for k in range(16)]
    ssems = (ssem0, ssem1)

    def _fire_gather(s, p):
        pltpu.async_copy(table_hbm.at[idx_v.at[s]], g_v.at[p], gsem)

    def _wait_gather(s, p):
        pltpu.make_async_copy(table_hbm.at[idx_v.at[s]], g_v.at[p],
                              gsem).wait()

    def _fire_store(s, p):
        pltpu.async_copy(gt_v.at[p], out_hbm.at[s, :, pl.ds(b0, _BPW)],
                         ssems[p])

    def _wait_store(s, p):
        pltpu.make_async_copy(gt_v.at[p], out_hbm.at[s, :, pl.ds(b0, _BPW)],
                              ssems[p]).wait()

    def _transpose(p):
        def _tb(t, c2):
            b0 = t * 16
            rowv = iota16 + b0
            for d0 in range(0, _D, 16):
                loaded = []
                for k in range(16):
                    colv = diag[k] + d0
                    loaded.append(
                        (colv, plsc.load_gather(g_v.at[p], [rowv, colv])))
                for colv, v in loaded:
                    plsc.store_scatter(gt_v.at[p], [colv, rowv], v)
            return c2

        lax.fori_loop(0, _BPW // 16, _tb, 0)

    _fire_gather(0, 0)

    def _pipe(t, carry):
        for u in range(2):
            s = 2 * t + u
            p = u
            _wait_gather(s, p)

            @pl.when(s + 1 < _SLOTS)
            def _():
                _fire_gather(s + 1, 1 - p)

            @pl.when(s >= 2)
            def _():
                _wait_store(s - 2, p)

            _transpose(p)
            _fire_store(s, p)
        return carry

    lax.fori_loop(0, _SLOTS // 2, _pipe, 0)
    _wait_store(_SLOTS - 2, 0)
    _wait_store(_SLOTS - 1, 1)


def kernel(blueprint_indices, table):
    idx_t = blueprint_indices.T.astype(jnp.int32)
    p = _sc_gather(idx_t, table)
    return lax.transpose(p, (2, 0, 1))
